# Initial kernel scaffold; baseline (speedup 1.0000x reference)
import jax
import jax.numpy as jnp
from jax.experimental import pallas as pl


def kernel(input_ids, table):
    raise NotImplementedError("write your pallas kernel here")


# speedup vs baseline: 1.8749x; 1.8749x over previous
import functools

import jax
import jax.numpy as jnp
from jax import lax
from jax.experimental import pallas as pl
from jax.experimental.pallas import tpu as pltpu
from jax.experimental.pallas import tpu_sc as plsc

VOCAB_SIZE = 1000000
WIDTH = 64
BATCH = 16384
HIST = 50

NC = 2
NS = 16
NW = NC * NS

TOTAL = BATCH * HIST
PER_W = TOTAL // NW
G = 128
S = 5
CHUNK = S * G
NCHUNK = PER_W // CHUNK
NSTEP = PER_W // G


def _embed_kernel(ids_hbm, table_hbm, out_hbm, idx_v, rows0, rows1, sem0, sem1):
    wid = lax.axis_index("s") * NC + lax.axis_index("c")
    base = wid * PER_W

    pltpu.sync_copy(ids_hbm.at[wid], idx_v)

    def fire(buf, sem, c):
        for s in range(S):
            pltpu.async_copy(
                table_hbm.at[idx_v.at[c * S + s]],
                buf.at[pl.ds(s * G, G)],
                sem,
            )

    def drain(buf, sem):
        pltpu.make_async_copy(table_hbm.at[pl.ds(0, CHUNK)], buf, sem).wait()

    def write(buf, c):
        pltpu.sync_copy(buf, out_hbm.at[pl.ds(base + c * CHUNK, CHUNK)])

    fire(rows0, sem0, 0)

    def body(g, carry):
        c0 = 2 * g
        fire(rows1, sem1, c0 + 1)
        drain(rows0, sem0)
        write(rows0, c0)
        fire(rows0, sem0, c0 + 2)
        drain(rows1, sem1)
        write(rows1, c0 + 1)
        return carry

    lax.fori_loop(0, NCHUNK // 2 - 1, body, 0)

    c0 = NCHUNK - 2
    fire(rows1, sem1, c0 + 1)
    drain(rows0, sem0)
    write(rows0, c0)
    drain(rows1, sem1)
    write(rows1, c0 + 1)


@jax.jit
def _embed(ids3, table):
    mesh = plsc.VectorSubcoreMesh(core_axis_name="c", subcore_axis_name="s")
    out = pl.kernel(
        _embed_kernel,
        out_type=jax.ShapeDtypeStruct((TOTAL, WIDTH), jnp.float32),
        mesh=mesh,
        scratch_types=[
            pltpu.VMEM((NSTEP, G), jnp.int32),
            pltpu.VMEM((CHUNK, WIDTH), jnp.float32),
            pltpu.VMEM((CHUNK, WIDTH), jnp.float32),
            pltpu.SemaphoreType.DMA,
            pltpu.SemaphoreType.DMA,
        ],
        compiler_params=pltpu.CompilerParams(use_tc_tiling_on_sc=False),
    )(ids3, table)
    return out


def kernel(input_ids, table):
    ids3 = input_ids.reshape(NW, NSTEP, G).astype(jnp.int32)
    out = _embed(ids3, table)
    return out.reshape(BATCH, HIST, WIDTH)
